# baseline (speedup 1.0000x reference)
import functools

import jax
import jax.numpy as jnp
from jax import lax
from jax.experimental import pallas as pl
from jax.experimental.pallas import tpu as pltpu
from jax.experimental.pallas import tpu_sc as plsc

N = 10000
D = 128
G = 64
NC = 2
NS = 16
L = 16
NW = NC * NS
CH = 128
AR = 10240


def _sc_mesh():
    return plsc.VectorSubcoreMesh(
        core_axis_name="c", subcore_axis_name="s", num_cores=NC, num_subcores=NS
    )


@functools.lru_cache(maxsize=None)
def _build_deg_kernel(ew: int):
    def body(dst_hbm, zrow_hbm, deg_out, dst_v, deg_v):
        c = lax.axis_index("c")
        s = lax.axis_index("s")
        wid = c * NS + s
        pltpu.sync_copy(zrow_hbm, deg_v)
        pltpu.sync_copy(dst_hbm.at[wid], dst_v)
        ones = jnp.ones((L,), jnp.float32)

        def step(i, carry):
            idx = dst_v[pl.ds(i * L, L)]
            plsc.addupdate_scatter(deg_v, [idx], ones)
            return carry

        lax.fori_loop(0, ew // L, step, 0)
        pltpu.sync_copy(deg_v, deg_out.at[wid])

    return pl.kernel(
        body,
        out_type=jax.ShapeDtypeStruct((NW, AR), jnp.float32),
        mesh=_sc_mesh(),
        scratch_types=[
            pltpu.VMEM((ew,), jnp.int32),
            pltpu.VMEM((AR,), jnp.float32),
        ],
        compiler_params=pltpu.CompilerParams(needs_layout_passes=False),
    )


NPH = 2


@functools.lru_cache(maxsize=None)
def _build_scatter_kernel(nch0: int, nch1: int):
    assert nch0 % (2 * NPH) == 0 and nch1 % (2 * NPH) == 0
    nph_max = max(nch0, nch1) // NPH

    def body(hp_hbm, src_hbm, dst_hbm, zeros_hbm, out_hbm, src_v, dst_v, rows0,
             rows1, acc_sh, s0, s1):
        c = lax.axis_index("c")
        s = lax.axis_index("s")
        wid = c * NS + s
        zrows = AR // NS
        pltpu.sync_copy(zeros_hbm, acc_sh.at[pl.ds(s * zrows, zrows)])
        plsc.subcore_barrier()

        ncp = jnp.where(c == 0, nch0 // NPH, nch1 // NPH)
        wait0 = pltpu.make_async_copy(rows0, acc_sh.at[pl.ds(0, CH)], s0)
        wait1 = pltpu.make_async_copy(rows1, acc_sh.at[pl.ds(0, CH)], s1)

        for ph in range(NPH):
            pltpu.sync_copy(src_hbm.at[wid, ph], src_v)
            pltpu.sync_copy(dst_hbm.at[wid, ph], dst_v)
            pltpu.sync_copy(hp_hbm.at[src_v.at[0]], rows0)

            def step(i, carry):
                j = 2 * i
                pltpu.async_copy(rows0, acc_sh.at[dst_v.at[j]], s0, add=True)
                pltpu.sync_copy(hp_hbm.at[src_v.at[j + 1]], rows1)
                wait0.wait()
                pltpu.async_copy(rows1, acc_sh.at[dst_v.at[j + 1]], s1, add=True)
                pltpu.sync_copy(hp_hbm.at[src_v.at[j + 2]], rows0)
                wait1.wait()
                return carry

            lax.fori_loop(0, ncp // 2, step, 0)

        plsc.subcore_barrier()
        opr = AR // NS
        pltpu.sync_copy(
            acc_sh.at[pl.ds(s * opr, opr)], out_hbm.at[c, pl.ds(s * opr, opr)]
        )

    return pl.kernel(
        body,
        out_type=jax.ShapeDtypeStruct((NC, AR, D), jnp.float32),
        mesh=_sc_mesh(),
        scratch_types=[
            pltpu.VMEM((nph_max + 1, CH), jnp.int32),
            pltpu.VMEM((nph_max + 1, CH), jnp.int32),
            pltpu.VMEM((CH, D), jnp.float32),
            pltpu.VMEM((CH, D), jnp.float32),
            pltpu.VMEM_SHARED((AR, D), jnp.float32),
            pltpu.SemaphoreType.DMA,
            pltpu.SemaphoreType.DMA,
        ],
    )


def _tc_dense1(deg_ref, x_ref, w_ref, dis_ref, hp_ref):
    deg = jnp.sum(deg_ref[...], axis=1, keepdims=True)[:N] + 1.0
    dis = lax.rsqrt(deg)
    dis_ref[...] = dis
    hp_ref[...] = jnp.dot(x_ref[...], w_ref[...], preferred_element_type=jnp.float32) * dis


def _tc_dense2(acc_ref, hp_ref, dis_ref, b_ref, w_ref, out_ref):
    a = acc_ref[...]
    agg = a[0, :N] + a[1, :N] + hp_ref[...]
    h = jnp.maximum(dis_ref[...] * agg + b_ref[...], 0.0)
    out_ref[...] = (
        jnp.dot(h, w_ref[...], preferred_element_type=jnp.float32) * dis_ref[...]
    )


def _tc_dense3(acc_ref, hp_ref, dis_ref, b_ref, batch_ref, wd_ref, bd_ref,
               scores_ref, pooled_ref):
    a = acc_ref[...]
    agg = a[0, :N] + a[1, :N] + hp_ref[...]
    h = jnp.maximum(dis_ref[...] * agg + b_ref[...], 0.0)
    gids = lax.broadcasted_iota(jnp.int32, (N, G), 1)
    maskf = jnp.where(batch_ref[...] == gids, 1.0, 0.0)
    dn = (((0,), (0,)), ((), ()))
    sums = lax.dot_general(maskf, h, dn, preferred_element_type=jnp.float32)
    cnts = lax.dot_general(
        maskf, jnp.ones((N, 1), jnp.float32), dn, preferred_element_type=jnp.float32
    )
    pooled = sums / jnp.maximum(cnts, 1.0)
    pooled_ref[...] = pooled
    scores_ref[...] = (
        jnp.dot(pooled, wd_ref[...], preferred_element_type=jnp.float32) + bd_ref[...]
    )


def kernel(x, edge_index, batch, W1, b1, W2, b2, Wd, bd):
    e = edge_index.shape[1]
    F0 = 0.60
    m = 2 * NPH
    nch0 = max(m, int(round(e * F0 / (NS * CH * m))) * m)
    e0 = nch0 * CH * NS
    nch1 = -(-(e - e0) // (NS * CH * m)) * m
    nph_max = max(nch0, nch1) // NPH
    cap = NS * (nch0 + nch1) * CH
    pad = cap - e
    src = jnp.concatenate([edge_index[0], jnp.zeros((pad,), jnp.int32)])
    dst = jnp.concatenate([edge_index[1], jnp.full((pad,), N, jnp.int32)])

    def _split(a, padval):
        out = []
        for blk, nch in ((a[:e0], nch0), (a[e0:], nch1)):
            b = blk.reshape(NS, NPH, nch // NPH, CH)
            fill = jnp.full(
                (NS, NPH, nph_max + 1 - nch // NPH, CH), padval, jnp.int32)
            out.append(jnp.concatenate([b, fill], axis=2))
        return jnp.concatenate(out, axis=0)

    src3 = _split(src, 0)
    dst3 = _split(dst, N)

    ew = NPH * (nph_max + 1) * CH
    dst2 = dst3.reshape(NW, ew)
    zrow = jnp.zeros((AR,), jnp.float32)
    zeros_m = jnp.zeros((AR // NS, D), jnp.float32)

    deg_parts = _build_deg_kernel(ew)(dst2, zrow)
    deg_t = jnp.transpose(deg_parts)

    dense1 = pl.pallas_call(
        _tc_dense1,
        out_shape=(
            jax.ShapeDtypeStruct((N, 1), jnp.float32),
            jax.ShapeDtypeStruct((N, D), jnp.float32),
        ),
    )
    dis, hp1 = dense1(deg_t, x, W1)

    scatter = _build_scatter_kernel(nch0, nch1)
    acc1 = scatter(hp1, src3, dst3, zeros_m)

    dense2 = pl.pallas_call(
        _tc_dense2,
        out_shape=jax.ShapeDtypeStruct((N, D), jnp.float32),
    )
    hp2 = dense2(acc1, hp1, dis, b1.reshape(1, D), W2)

    acc2 = scatter(hp2, src3, dst3, zeros_m)

    dense3 = pl.pallas_call(
        _tc_dense3,
        out_shape=(
            jax.ShapeDtypeStruct((G, D), jnp.float32),
            jax.ShapeDtypeStruct((G, D), jnp.float32),
        ),
    )
    scores, pooled = dense3(
        acc2, hp2, dis, b2.reshape(1, D),
        batch.reshape(N, 1), Wd, bd.reshape(1, D),
    )
    return (scores, pooled)


# speedup vs baseline: 3.2248x; 3.2248x over previous
import functools

import jax
import jax.numpy as jnp
from jax import lax
from jax.experimental import pallas as pl
from jax.experimental.pallas import tpu as pltpu
from jax.experimental.pallas import tpu_sc as plsc

N = 10000
D = 128
G = 64
NC = 2
NS = 16
L = 16
NW = NC * NS
CH = 128
AR = 10240


def _sc_mesh():
    return plsc.VectorSubcoreMesh(
        core_axis_name="c", subcore_axis_name="s", num_cores=NC, num_subcores=NS
    )


@functools.lru_cache(maxsize=None)
def _build_deg_kernel(ew: int):
    def body(dst_hbm, zrow_hbm, deg_out, dst_v, deg_v):
        c = lax.axis_index("c")
        s = lax.axis_index("s")
        wid = c * NS + s
        pltpu.sync_copy(zrow_hbm, deg_v)
        pltpu.sync_copy(dst_hbm.at[wid], dst_v)
        ones = jnp.ones((L,), jnp.float32)

        def step(i, carry):
            idx = dst_v[pl.ds(i * L, L)]
            plsc.addupdate_scatter(deg_v, [idx], ones)
            return carry

        lax.fori_loop(0, ew // L, step, 0)
        pltpu.sync_copy(deg_v, deg_out.at[wid])

    return pl.kernel(
        body,
        out_type=jax.ShapeDtypeStruct((NW, AR), jnp.float32),
        mesh=_sc_mesh(),
        scratch_types=[
            pltpu.VMEM((ew,), jnp.int32),
            pltpu.VMEM((AR,), jnp.float32),
        ],
        compiler_params=pltpu.CompilerParams(needs_layout_passes=False),
    )


@functools.lru_cache(maxsize=None)
def _build_scatter_kernel(nch0: int, nch1: int):
    nch_max = max(nch0, nch1)

    def body(hp_hbm, src_hbm, dst_hbm, zeros_hbm, out_hbm, src_v, dst_v, rows_v,
             acc_sh):
        c = lax.axis_index("c")
        s = lax.axis_index("s")
        wid = c * NS + s
        zrows = AR // NS
        pltpu.sync_copy(zeros_hbm, acc_sh.at[pl.ds(s * zrows, zrows)])
        pltpu.sync_copy(src_hbm.at[wid], src_v)
        pltpu.sync_copy(dst_hbm.at[wid], dst_v)
        plsc.subcore_barrier()

        nc = jnp.where(c == 0, nch0, nch1)

        def step(j, carry):
            pltpu.sync_copy(hp_hbm.at[src_v.at[j]], rows_v)
            pltpu.sync_copy(rows_v, acc_sh.at[dst_v.at[j]], add=True)
            return carry

        lax.fori_loop(0, nc, step, 0)
        plsc.subcore_barrier()
        opr = AR // NS
        pltpu.sync_copy(
            acc_sh.at[pl.ds(s * opr, opr)], out_hbm.at[c, pl.ds(s * opr, opr)]
        )

    return pl.kernel(
        body,
        out_type=jax.ShapeDtypeStruct((NC, AR, D), jnp.float32),
        mesh=_sc_mesh(),
        scratch_types=[
            pltpu.VMEM((nch_max, CH), jnp.int32),
            pltpu.VMEM((nch_max, CH), jnp.int32),
            pltpu.VMEM((CH, D), jnp.float32),
            pltpu.VMEM_SHARED((AR, D), jnp.float32),
        ],
    )


def _tc_dense1(deg_ref, x_ref, w_ref, dis_ref, hp_ref):
    deg = jnp.sum(deg_ref[...], axis=1, keepdims=True)[:N] + 1.0
    dis = lax.rsqrt(deg)
    dis_ref[...] = dis
    hp_ref[...] = jnp.dot(x_ref[...], w_ref[...], preferred_element_type=jnp.float32) * dis


def _tc_dense2(acc_ref, hp_ref, dis_ref, b_ref, w_ref, out_ref):
    a = acc_ref[...]
    agg = a[0, :N] + a[1, :N] + hp_ref[...]
    h = jnp.maximum(dis_ref[...] * agg + b_ref[...], 0.0)
    out_ref[...] = (
        jnp.dot(h, w_ref[...], preferred_element_type=jnp.float32) * dis_ref[...]
    )


def _tc_dense3(acc_ref, hp_ref, dis_ref, b_ref, batch_ref, wd_ref, bd_ref,
               scores_ref, pooled_ref):
    a = acc_ref[...]
    agg = a[0, :N] + a[1, :N] + hp_ref[...]
    h = jnp.maximum(dis_ref[...] * agg + b_ref[...], 0.0)
    gids = lax.broadcasted_iota(jnp.int32, (N, G), 1)
    maskf = jnp.where(batch_ref[...] == gids, 1.0, 0.0)
    dn = (((0,), (0,)), ((), ()))
    sums = lax.dot_general(maskf, h, dn, preferred_element_type=jnp.float32)
    cnts = lax.dot_general(
        maskf, jnp.ones((N, 1), jnp.float32), dn, preferred_element_type=jnp.float32
    )
    pooled = sums / jnp.maximum(cnts, 1.0)
    pooled_ref[...] = pooled
    scores_ref[...] = (
        jnp.dot(pooled, wd_ref[...], preferred_element_type=jnp.float32) + bd_ref[...]
    )


def kernel(x, edge_index, batch, W1, b1, W2, b2, Wd, bd):
    e = edge_index.shape[1]
    F0 = 0.63
    nch0 = max(1, int(round(e * F0 / (NS * CH))))
    e0 = nch0 * CH * NS
    nch1 = -(-(e - e0) // (NS * CH))
    nch_max = max(nch0, nch1)
    cap = NS * (nch0 + nch1) * CH
    pad = cap - e
    src = jnp.concatenate([edge_index[0], jnp.zeros((pad,), jnp.int32)])
    dst = jnp.concatenate([edge_index[1], jnp.full((pad,), N, jnp.int32)])

    def _split(a, padval):
        a0 = a[:e0].reshape(NS, nch0, CH)
        a1 = a[e0:].reshape(NS, nch1, CH)
        a0 = jnp.concatenate(
            [a0, jnp.full((NS, nch_max - nch0, CH), padval, jnp.int32)], axis=1)
        a1 = jnp.concatenate(
            [a1, jnp.full((NS, nch_max - nch1, CH), padval, jnp.int32)], axis=1)
        return jnp.concatenate([a0, a1], axis=0)

    src3 = _split(src, 0)
    dst3 = _split(dst, N)

    ew = nch_max * CH
    dst2 = dst3.reshape(NW, ew)
    zrow = jnp.zeros((AR,), jnp.float32)
    zeros_m = jnp.zeros((AR // NS, D), jnp.float32)

    deg_parts = _build_deg_kernel(ew)(dst2, zrow)
    deg_t = jnp.transpose(deg_parts)

    dense1 = pl.pallas_call(
        _tc_dense1,
        out_shape=(
            jax.ShapeDtypeStruct((N, 1), jnp.float32),
            jax.ShapeDtypeStruct((N, D), jnp.float32),
        ),
    )
    dis, hp1 = dense1(deg_t, x, W1)

    scatter = _build_scatter_kernel(nch0, nch1)
    acc1 = scatter(hp1, src3, dst3, zeros_m)

    dense2 = pl.pallas_call(
        _tc_dense2,
        out_shape=jax.ShapeDtypeStruct((N, D), jnp.float32),
    )
    hp2 = dense2(acc1, hp1, dis, b1.reshape(1, D), W2)

    acc2 = scatter(hp2, src3, dst3, zeros_m)

    dense3 = pl.pallas_call(
        _tc_dense3,
        out_shape=(
            jax.ShapeDtypeStruct((G, D), jnp.float32),
            jax.ShapeDtypeStruct((G, D), jnp.float32),
        ),
    )
    scores, pooled = dense3(
        acc2, hp2, dis, b2.reshape(1, D),
        batch.reshape(N, 1), Wd, bd.reshape(1, D),
    )
    return (scores, pooled)


# speedup vs baseline: 3.3282x; 1.0321x over previous
import functools

import jax
import jax.numpy as jnp
from jax import lax
from jax.experimental import pallas as pl
from jax.experimental.pallas import tpu as pltpu
from jax.experimental.pallas import tpu_sc as plsc

N = 10000
D = 128
G = 64
NC = 2
NS = 16
L = 16
NW = NC * NS
CH = 128
AR = 10240


def _sc_mesh():
    return plsc.VectorSubcoreMesh(
        core_axis_name="c", subcore_axis_name="s", num_cores=NC, num_subcores=NS
    )


@functools.lru_cache(maxsize=None)
def _build_deg_kernel(ew: int):
    def body(dst_hbm, zrow_hbm, deg_out, dst_v, deg_v):
        c = lax.axis_index("c")
        s = lax.axis_index("s")
        wid = c * NS + s
        pltpu.sync_copy(zrow_hbm, deg_v)
        pltpu.sync_copy(dst_hbm.at[wid], dst_v)
        ones = jnp.ones((L,), jnp.float32)

        def step(i, carry):
            idx = dst_v[pl.ds(i * L, L)]
            plsc.addupdate_scatter(deg_v, [idx], ones)
            return carry

        lax.fori_loop(0, ew // L, step, 0)
        pltpu.sync_copy(deg_v, deg_out.at[wid])

    return pl.kernel(
        body,
        out_type=jax.ShapeDtypeStruct((NW, AR), jnp.float32),
        mesh=_sc_mesh(),
        scratch_types=[
            pltpu.VMEM((ew,), jnp.int32),
            pltpu.VMEM((AR,), jnp.float32),
        ],
        compiler_params=pltpu.CompilerParams(needs_layout_passes=False),
    )


@functools.lru_cache(maxsize=None)
def _build_scatter_kernel(nch0: int, nch1: int):
    nch_max = max(nch0, nch1)

    def body(hp_hbm, src_hbm, dst_hbm, zeros_hbm, out_hbm, src_v, dst_v, rows_v,
             acc_sh):
        c = lax.axis_index("c")
        s = lax.axis_index("s")
        wid = c * NS + s
        zrows = AR // NS
        pltpu.sync_copy(zeros_hbm, acc_sh.at[pl.ds(s * zrows, zrows)])
        pltpu.sync_copy(src_hbm.at[wid], src_v)
        pltpu.sync_copy(dst_hbm.at[wid], dst_v)
        plsc.subcore_barrier()

        nc = jnp.where(c == 0, nch0, nch1)

        def step(j, carry):
            pltpu.sync_copy(hp_hbm.at[src_v.at[j]], rows_v)
            pltpu.sync_copy(rows_v, acc_sh.at[dst_v.at[j]], add=True)
            return carry

        lax.fori_loop(0, nc, step, 0)
        plsc.subcore_barrier()
        opr = AR // NS
        pltpu.sync_copy(
            acc_sh.at[pl.ds(s * opr, opr)], out_hbm.at[c, pl.ds(s * opr, opr)]
        )

    return pl.kernel(
        body,
        out_type=jax.ShapeDtypeStruct((NC, AR, D), jnp.float32),
        mesh=_sc_mesh(),
        scratch_types=[
            pltpu.VMEM((nch_max, CH), jnp.int32),
            pltpu.VMEM((nch_max, CH), jnp.int32),
            pltpu.VMEM((CH, D), jnp.float32),
            pltpu.VMEM_SHARED((AR, D), jnp.float32),
        ],
    )


def _tc_dense1(deg_ref, x_ref, w_ref, dis_ref, hp_ref):
    deg = jnp.sum(deg_ref[...], axis=1, keepdims=True)[:N] + 1.0
    dis = lax.rsqrt(deg)
    dis_ref[...] = dis
    hp_ref[...] = jnp.dot(x_ref[...], w_ref[...], preferred_element_type=jnp.float32) * dis


def _tc_dense2(acc_ref, hp_ref, dis_ref, b_ref, w_ref, out_ref):
    a = acc_ref[...]
    agg = a[0, :N] + a[1, :N] + hp_ref[...]
    h = jnp.maximum(dis_ref[...] * agg + b_ref[...], 0.0)
    out_ref[...] = (
        jnp.dot(h, w_ref[...], preferred_element_type=jnp.float32) * dis_ref[...]
    )


def _tc_dense3(acc_ref, hp_ref, dis_ref, b_ref, batch_ref, wd_ref, bd_ref,
               scores_ref, pooled_ref):
    a = acc_ref[...]
    agg = a[0, :N] + a[1, :N] + hp_ref[...]
    h = jnp.maximum(dis_ref[...] * agg + b_ref[...], 0.0)
    gids = lax.broadcasted_iota(jnp.int32, (N, G), 1)
    maskf = jnp.where(batch_ref[...] == gids, 1.0, 0.0)
    dn = (((0,), (0,)), ((), ()))
    sums = lax.dot_general(maskf, h, dn, preferred_element_type=jnp.float32)
    cnts = lax.dot_general(
        maskf, jnp.ones((N, 1), jnp.float32), dn, preferred_element_type=jnp.float32
    )
    pooled = sums / jnp.maximum(cnts, 1.0)
    pooled_ref[...] = pooled
    scores_ref[...] = (
        jnp.dot(pooled, wd_ref[...], preferred_element_type=jnp.float32) + bd_ref[...]
    )


def kernel(x, edge_index, batch, W1, b1, W2, b2, Wd, bd):
    e = edge_index.shape[1]
    F0 = 0.57
    nch0 = max(1, int(round(e * F0 / (NS * CH))))
    e0 = nch0 * CH * NS
    nch1 = -(-(e - e0) // (NS * CH))
    nch_max = max(nch0, nch1)
    cap = NS * (nch0 + nch1) * CH
    pad = cap - e
    src = jnp.concatenate([edge_index[0], jnp.zeros((pad,), jnp.int32)])
    dst = jnp.concatenate([edge_index[1], jnp.full((pad,), N, jnp.int32)])

    def _split(a, padval):
        a0 = a[:e0].reshape(NS, nch0, CH)
        a1 = a[e0:].reshape(NS, nch1, CH)
        a0 = jnp.concatenate(
            [a0, jnp.full((NS, nch_max - nch0, CH), padval, jnp.int32)], axis=1)
        a1 = jnp.concatenate(
            [a1, jnp.full((NS, nch_max - nch1, CH), padval, jnp.int32)], axis=1)
        return jnp.concatenate([a0, a1], axis=0)

    src3 = _split(src, 0)
    dst3 = _split(dst, N)

    ew = nch_max * CH
    dst2 = dst3.reshape(NW, ew)
    zrow = jnp.zeros((AR,), jnp.float32)
    zeros_m = jnp.zeros((AR // NS, D), jnp.float32)

    deg_parts = _build_deg_kernel(ew)(dst2, zrow)
    deg_t = jnp.transpose(deg_parts)

    dense1 = pl.pallas_call(
        _tc_dense1,
        out_shape=(
            jax.ShapeDtypeStruct((N, 1), jnp.float32),
            jax.ShapeDtypeStruct((N, D), jnp.float32),
        ),
    )
    dis, hp1 = dense1(deg_t, x, W1)

    scatter = _build_scatter_kernel(nch0, nch1)
    acc1 = scatter(hp1, src3, dst3, zeros_m)

    dense2 = pl.pallas_call(
        _tc_dense2,
        out_shape=jax.ShapeDtypeStruct((N, D), jnp.float32),
    )
    hp2 = dense2(acc1, hp1, dis, b1.reshape(1, D), W2)

    acc2 = scatter(hp2, src3, dst3, zeros_m)

    dense3 = pl.pallas_call(
        _tc_dense3,
        out_shape=(
            jax.ShapeDtypeStruct((G, D), jnp.float32),
            jax.ShapeDtypeStruct((G, D), jnp.float32),
        ),
    )
    scores, pooled = dense3(
        acc2, hp2, dis, b2.reshape(1, D),
        batch.reshape(N, 1), Wd, bd.reshape(1, D),
    )
    return (scores, pooled)
